# baseline (speedup 1.0000x reference)
import jax
import jax.numpy as jnp
from jax.experimental import pallas as pl


def _bias_add_kernel(h_ref, b_ref, o_ref):
    o_ref[...] = h_ref[...] + b_ref[...]


def _cheb_layer(h, W, b, src, dst, w_edge, n_nodes):
    def lhat(t):
        return jax.ops.segment_sum(w_edge[:, None] * t[src], dst, num_segments=n_nodes)
    Tx0 = h
    out = Tx0 @ W[0]
    Tx1 = lhat(h)
    out = out + Tx1 @ W[1]
    for k in range(2, W.shape[0]):
        Tx2 = 2.0 * lhat(Tx1) - Tx0
        out = out + Tx2 @ W[k]
        Tx0, Tx1 = Tx1, Tx2
    bb = jnp.broadcast_to(b, out.shape)
    blk = 2000
    out = pl.pallas_call(
        _bias_add_kernel,
        grid=(out.shape[0] // blk,),
        in_specs=[
            pl.BlockSpec((blk, out.shape[1]), lambda i: (i, 0)),
            pl.BlockSpec((blk, out.shape[1]), lambda i: (i, 0)),
        ],
        out_specs=pl.BlockSpec((blk, out.shape[1]), lambda i: (i, 0)),
        out_shape=jax.ShapeDtypeStruct(out.shape, out.dtype),
    )(out, bb)
    return out


def kernel(x, edge_index, W1, b1, W2, b2, W3, b3, W4, b4, W5, b5, W6, b6, W7, b7, W8, b8):
    n_nodes = x.shape[0]
    src = edge_index[0]
    dst = edge_index[1]
    deg = jax.ops.segment_sum(jnp.ones((edge_index.shape[1],), dtype=jnp.float32), src, num_segments=n_nodes)
    dis = jnp.where(deg > 0, 1.0 / jnp.sqrt(jnp.where(deg > 0, deg, 1.0)), 0.0)
    w_edge = -dis[src] * dis[dst]
    Ws = [W1, W2, W3, W4, W5, W6, W7, W8]
    bs = [b1, b2, b3, b4, b5, b6, b7, b8]
    h = x
    for i in range(8):
        h = _cheb_layer(h, Ws[i], bs[i], src, dst, w_edge, n_nodes)
        if i < 7:
            h = jax.nn.relu(h)
    return h


# speedup vs baseline: 1.0410x; 1.0410x over previous
import functools

import jax
import jax.numpy as jnp
from jax.experimental import pallas as pl


def _combine_kernel(K, relu, *refs):
    txs = refs[:K]
    w_ref = refs[K]
    b_ref = refs[K + 1]
    o_ref = refs[K + 2]
    acc = jnp.dot(txs[0][...], w_ref[0], preferred_element_type=jnp.float32)
    for k in range(1, K):
        acc = acc + jnp.dot(txs[k][...], w_ref[k], preferred_element_type=jnp.float32)
    acc = acc + b_ref[...]
    if relu:
        acc = jnp.maximum(acc, 0.0)
    o_ref[...] = acc


def _combine(txs, W, b, relu):
    K, din, dout = W.shape
    n = txs[0].shape[0]
    blk = 2000
    grid = (n // blk,)
    in_specs = [pl.BlockSpec((blk, din), lambda i: (i, 0)) for _ in range(K)]
    in_specs.append(pl.BlockSpec((K, din, dout), lambda i: (0, 0, 0)))
    in_specs.append(pl.BlockSpec((1, dout), lambda i: (0, 0)))
    return pl.pallas_call(
        functools.partial(_combine_kernel, K, relu),
        grid=grid,
        in_specs=in_specs,
        out_specs=pl.BlockSpec((blk, dout), lambda i: (i, 0)),
        out_shape=jax.ShapeDtypeStruct((n, dout), jnp.float32),
    )(*txs, W, b.reshape(1, dout))


def kernel(x, edge_index, W1, b1, W2, b2, W3, b3, W4, b4, W5, b5, W6, b6, W7, b7, W8, b8):
    n_nodes = x.shape[0]
    src = edge_index[0]
    dst = edge_index[1]
    deg = jax.ops.segment_sum(
        jnp.ones((edge_index.shape[1],), dtype=jnp.float32), src, num_segments=n_nodes
    )
    dis = jnp.where(deg > 0, 1.0 / jnp.sqrt(jnp.where(deg > 0, deg, 1.0)), 0.0)
    w_edge = -dis[src] * dis[dst]

    perm = jnp.argsort(dst)
    src_s = src[perm]
    dst_s = dst[perm]
    w_s = w_edge[perm]
    w2_s = 2.0 * w_s

    def lhat(t, coef):
        v = coef[:, None] * t[src_s]
        return jax.ops.segment_sum(
            v, dst_s, num_segments=n_nodes, indices_are_sorted=True
        )

    Ws = [W1, W2, W3, W4, W5, W6, W7, W8]
    bs = [b1, b2, b3, b4, b5, b6, b7, b8]
    h = x
    for i in range(8):
        W = Ws[i]
        K = W.shape[0]
        txs = [h, lhat(h, w_s)]
        for _ in range(2, K):
            txs.append(lhat(txs[-1], w2_s) - txs[-2])
        h = _combine(txs, W, bs[i], relu=(i < 7))
    return h
